# Initial kernel scaffold; baseline (speedup 1.0000x reference)
import jax
import jax.numpy as jnp
from jax.experimental import pallas as pl


def kernel(patch):
    raise NotImplementedError("write your pallas kernel here")


# speedup vs baseline: 2.5591x; 2.5591x over previous
import functools

import numpy as np
import jax
import jax.numpy as jnp
from jax import lax
from jax.experimental import pallas as pl
from jax.experimental.pallas import tpu as pltpu, tpu_sc as plsc

_SIZE, _PATCH = 384, 16
_NUM = _SIZE // _PATCH
_P = _NUM * _NUM
_C = 32
_D = 768
_K = 8


def _static_indices():
    grid = np.arange(_P, dtype=np.int32).reshape(_NUM, _NUM)
    idx, inv = [], []
    for k in range(4):
        rot = np.rot90(grid, k=k, axes=(0, 1))
        for g in (rot, np.flip(rot, axis=1)):
            flat = g.flatten()
            idx.append(flat)
            inv.append(np.argsort(flat).astype(np.int32))
    return np.stack(idx), np.stack(inv)


_IDX, _ARGSORT = _static_indices()
_GIDX = (np.arange(_C, dtype=np.int32)[None, :, None] * _P
         + _IDX[:, None, :]).reshape(-1)

_B = _K * _C * _P
_NC, _NS = 2, 16
_NW = _NC * _NS
_BPW = _B // _NW
_R = 128
_NCHUNK = _BPW // _R


def _sc_gather(table, gidx):
    mesh = plsc.VectorSubcoreMesh(core_axis_name="c", subcore_axis_name="s")

    @functools.partial(
        pl.kernel,
        mesh=mesh,
        out_type=jax.ShapeDtypeStruct((_B, _D), jnp.float32),
        scratch_types=[
            pltpu.VMEM((_BPW,), jnp.int32),
            pltpu.VMEM((_R, _D), jnp.float32),
            pltpu.SemaphoreType.DMA,
        ],
    )
    def gather_kernel(table_hbm, gidx_hbm, out_hbm, idx_v, rows_v, sem):
        wid = lax.axis_index("s") * _NC + lax.axis_index("c")
        base = wid * _BPW
        pltpu.sync_copy(gidx_hbm.at[pl.ds(base, _BPW)], idx_v)

        def body(i, carry):
            off = i * _R
            pltpu.async_copy(
                table_hbm.at[idx_v.at[pl.ds(off, _R)]], rows_v, sem
            ).wait()
            pltpu.sync_copy(rows_v, out_hbm.at[pl.ds(base + off, _R)])
            return carry

        lax.fori_loop(0, _NCHUNK, body, 0)

    return gather_kernel(table, gidx)


def kernel(patch):
    table = patch.reshape(_C * _P, _D)
    out = _sc_gather(table, jnp.asarray(_GIDX))
    aug_tensor = out.reshape(_K, _C, _P, _D)
    argsort_tensor = jnp.asarray(_ARGSORT)
    perm = jnp.arange(_K, dtype=jnp.int32)
    return aug_tensor, argsort_tensor, perm


# speedup vs baseline: 2.6888x; 1.0507x over previous
import functools

import numpy as np
import jax
import jax.numpy as jnp
from jax import lax
from jax.experimental import pallas as pl
from jax.experimental.pallas import tpu as pltpu, tpu_sc as plsc

_SIZE, _PATCH = 384, 16
_NUM = _SIZE // _PATCH
_P = _NUM * _NUM
_C = 32
_D = 768
_K = 8


def _static_indices():
    grid = np.arange(_P, dtype=np.int32).reshape(_NUM, _NUM)
    idx, inv = [], []
    for k in range(4):
        rot = np.rot90(grid, k=k, axes=(0, 1))
        for g in (rot, np.flip(rot, axis=1)):
            flat = g.flatten()
            idx.append(flat)
            inv.append(np.argsort(flat).astype(np.int32))
    return np.stack(idx), np.stack(inv)


_IDX, _ARGSORT = _static_indices()
_GIDX = (np.arange(_C, dtype=np.int32)[None, :, None] * _P
         + _IDX[:, None, :]).reshape(-1)

_B = _K * _C * _P
_NC, _NS = 2, 16
_NW = _NC * _NS
_BPW = _B // _NW
_R = 64
_NCHUNK = _BPW // _R
_NPAIR = _NCHUNK // 2


def _sc_gather(table, gidx):
    mesh = plsc.VectorSubcoreMesh(core_axis_name="c", subcore_axis_name="s")

    @functools.partial(
        pl.kernel,
        mesh=mesh,
        out_type=jax.ShapeDtypeStruct((_B, _D), jnp.float32),
        scratch_types=[
            pltpu.VMEM((_BPW,), jnp.int32),
            pltpu.VMEM((_R, _D), jnp.float32),
            pltpu.VMEM((_R, _D), jnp.float32),
            pltpu.SemaphoreType.DMA,
            pltpu.SemaphoreType.DMA,
            pltpu.SemaphoreType.DMA,
            pltpu.SemaphoreType.DMA,
        ],
    )
    def gather_kernel(table_hbm, gidx_hbm, out_hbm, idx_v,
                      rows0, rows1, gsem0, gsem1, ssem0, ssem1):
        wid = lax.axis_index("s") * _NC + lax.axis_index("c")
        base = wid * _BPW
        pltpu.sync_copy(gidx_hbm.at[pl.ds(base, _BPW)], idx_v)

        bufs = (rows0, rows1)
        gsems = (gsem0, gsem1)
        ssems = (ssem0, ssem1)

        def gather_desc(i, b):
            return pltpu.make_async_copy(
                table_hbm.at[idx_v.at[pl.ds(i * _R, _R)]], bufs[b], gsems[b])

        def scatter_desc(i, b):
            return pltpu.make_async_copy(
                bufs[b], out_hbm.at[pl.ds(base + i * _R, _R)], ssems[b])

        gather_desc(0, 0).start()

        def pair(j, carry):
            i0 = 2 * j
            @pl.when(j > 0)
            def _():
                scatter_desc(i0 - 1, 1).wait()
            gather_desc(i0 + 1, 1).start()
            gather_desc(i0, 0).wait()
            scatter_desc(i0, 0).start()
            scatter_desc(i0, 0).wait()
            @pl.when(j < _NPAIR - 1)
            def _():
                gather_desc(i0 + 2, 0).start()
            gather_desc(i0 + 1, 1).wait()
            scatter_desc(i0 + 1, 1).start()
            return carry

        lax.fori_loop(0, _NPAIR, pair, 0)
        scatter_desc(_NCHUNK - 1, 1).wait()

    return gather_kernel(table, gidx)


def kernel(patch):
    table = patch.reshape(_C * _P, _D)
    out = _sc_gather(table, jnp.asarray(_GIDX))
    aug_tensor = out.reshape(_K, _C, _P, _D)
    argsort_tensor = jnp.asarray(_ARGSORT)
    perm = jnp.arange(_K, dtype=jnp.int32)
    return aug_tensor, argsort_tensor, perm


# speedup vs baseline: 4.3807x; 1.6292x over previous
import functools

import numpy as np
import jax
import jax.numpy as jnp
from jax import lax
from jax.experimental import pallas as pl
from jax.experimental.pallas import tpu as pltpu, tpu_sc as plsc

_SIZE, _PATCH = 384, 16
_NUM = _SIZE // _PATCH
_P = _NUM * _NUM
_C = 32
_D = 768
_K = 8


def _static_indices():
    grid = np.arange(_P, dtype=np.int32).reshape(_NUM, _NUM)
    idx, inv = [], []
    for k in range(4):
        rot = np.rot90(grid, k=k, axes=(0, 1))
        for g in (rot, np.flip(rot, axis=1)):
            flat = g.flatten()
            idx.append(flat)
            inv.append(np.argsort(flat).astype(np.int32))
    return np.stack(idx), np.stack(inv)


_IDX, _ARGSORT = _static_indices()

_B = _K * _C * _P
_NC, _NS = 2, 16
_NW = _NC * _NS
_Q = 48
_NQ = _P // _Q
_NPAIR = _NQ // 2

_SIDX = (np.arange(_NW, dtype=np.int32)[:, None, None, None] * _P
         + np.arange(_K, dtype=np.int32)[None, None, :, None] * (_C * _P)
         + _ARGSORT.reshape(1, _K, _NQ, _Q).transpose(0, 2, 1, 3)
         ).reshape(_NW, _NQ * _K, _Q).astype(np.int32)


def _sc_augment(table, sidx):
    mesh = plsc.VectorSubcoreMesh(core_axis_name="c", subcore_axis_name="s")

    @functools.partial(
        pl.kernel,
        mesh=mesh,
        out_type=jax.ShapeDtypeStruct((_B, _D), jnp.float32),
        scratch_types=[
            pltpu.VMEM((_NQ * _K, _Q), jnp.int32),
            pltpu.VMEM((_Q, _D), jnp.float32),
            pltpu.VMEM((_Q, _D), jnp.float32),
            pltpu.SemaphoreType.DMA,
        ],
    )
    def aug_kernel(table_hbm, sidx_hbm, out_hbm, sidx_v, buf0, buf1, ssem):
        wid = lax.axis_index("s") * _NC + lax.axis_index("c")
        pltpu.sync_copy(sidx_hbm.at[wid], sidx_v)
        in_base = wid * _P
        bufs = (buf0, buf1)

        def drain_one():
            pltpu.make_async_copy(table_hbm.at[pl.ds(0, _Q)], buf0, ssem).wait()

        def chunk_step(q, b):
            @pl.when(q >= 2)
            def _():
                for _ in range(_K):
                    drain_one()
            pltpu.sync_copy(table_hbm.at[pl.ds(in_base + q * _Q, _Q)], bufs[b])
            for kk in range(_K):
                pltpu.make_async_copy(
                    bufs[b], out_hbm.at[sidx_v.at[q * _K + kk]], ssem
                ).start()

        def pair(j, carry):
            chunk_step(2 * j, 0)
            chunk_step(2 * j + 1, 1)
            return carry

        lax.fori_loop(0, _NPAIR, pair, 0)
        for _ in range(2 * _K):
            drain_one()

    return aug_kernel(table, sidx)


def kernel(patch):
    table = patch.reshape(_C * _P, _D)
    out = _sc_augment(table, jnp.asarray(_SIDX))
    aug_tensor = out.reshape(_K, _C, _P, _D)
    argsort_tensor = jnp.asarray(_ARGSORT)
    perm = jnp.arange(_K, dtype=jnp.int32)
    return aug_tensor, argsort_tensor, perm


# speedup vs baseline: 4.4866x; 1.0242x over previous
import functools

import numpy as np
import jax
import jax.numpy as jnp
from jax import lax
from jax.experimental import pallas as pl
from jax.experimental.pallas import tpu as pltpu, tpu_sc as plsc

_SIZE, _PATCH = 384, 16
_NUM = _SIZE // _PATCH
_P = _NUM * _NUM
_C = 32
_D = 768
_K = 8


def _static_indices():
    grid = np.arange(_P, dtype=np.int32).reshape(_NUM, _NUM)
    idx, inv = [], []
    for k in range(4):
        rot = np.rot90(grid, k=k, axes=(0, 1))
        for g in (rot, np.flip(rot, axis=1)):
            flat = g.flatten()
            idx.append(flat)
            inv.append(np.argsort(flat).astype(np.int32))
    return np.stack(idx), np.stack(inv)


_IDX, _ARGSORT = _static_indices()

_B = _K * _C * _P
_NC, _NS = 2, 16
_NW = _NC * _NS
_Q = 72
_NQ = _P // _Q
_NPAIR = _NQ // 2

_SIDX = (np.arange(_NW, dtype=np.int32)[:, None, None, None] * _P
         + np.arange(_K, dtype=np.int32)[None, None, :, None] * (_C * _P)
         + _ARGSORT.reshape(1, _K, _NQ, _Q).transpose(0, 2, 1, 3)
         ).reshape(_NW, _NQ * _K, _Q).astype(np.int32)


def _sc_augment(table, sidx):
    mesh = plsc.VectorSubcoreMesh(core_axis_name="c", subcore_axis_name="s")

    @functools.partial(
        pl.kernel,
        mesh=mesh,
        out_type=jax.ShapeDtypeStruct((_B, _D), jnp.float32),
        scratch_types=[
            pltpu.VMEM((_NQ * _K, _Q), jnp.int32),
            pltpu.VMEM((_Q, _D), jnp.float32),
            pltpu.VMEM((_Q, _D), jnp.float32),
            pltpu.SemaphoreType.DMA,
        ],
    )
    def aug_kernel(table_hbm, sidx_hbm, out_hbm, sidx_v, buf0, buf1, ssem):
        wid = lax.axis_index("s") * _NC + lax.axis_index("c")
        pltpu.sync_copy(sidx_hbm.at[wid], sidx_v)
        in_base = wid * _P
        bufs = (buf0, buf1)

        def drain_one():
            pltpu.make_async_copy(table_hbm.at[pl.ds(0, _Q)], buf0, ssem).wait()

        def chunk_step(q, b):
            @pl.when(q >= 2)
            def _():
                for _ in range(_K):
                    drain_one()
            pltpu.sync_copy(table_hbm.at[pl.ds(in_base + q * _Q, _Q)], bufs[b])
            for kk in range(_K):
                pltpu.make_async_copy(
                    bufs[b], out_hbm.at[sidx_v.at[q * _K + kk]], ssem
                ).start()

        def pair(j, carry):
            chunk_step(2 * j, 0)
            chunk_step(2 * j + 1, 1)
            return carry

        lax.fori_loop(0, _NPAIR, pair, 0)
        for _ in range(2 * _K):
            drain_one()

    return aug_kernel(table, sidx)


def kernel(patch):
    table = patch.reshape(_C * _P, _D)
    out = _sc_augment(table, jnp.asarray(_SIDX))
    aug_tensor = out.reshape(_K, _C, _P, _D)
    argsort_tensor = jnp.asarray(_ARGSORT)
    perm = jnp.arange(_K, dtype=jnp.int32)
    return aug_tensor, argsort_tensor, perm
